# Initial kernel scaffold; baseline (speedup 1.0000x reference)
import jax
import jax.numpy as jnp
from jax.experimental import pallas as pl


def kernel(in_feat, edge_index, W1, b1, W2, b2):
    raise NotImplementedError("write your pallas kernel here")


# speedup vs baseline: 10.0756x; 10.0756x over previous
import functools

import jax
import jax.numpy as jnp
from jax import lax
from jax.experimental import pallas as pl
from jax.experimental.pallas import tpu as pltpu
from jax.experimental.pallas import tpu_sc as plsc

N = 10000
NN = 10112
E = 320000
F = 128
FH = F // 2
NCLS = 40
NC = 2
NS = 16
L = 16
CH = 128
PT = E // NS
NCHK = 157
PTP = NCHK * CH
NBLK = NN // 128
PAD_NODE = N
CPT = NN // NS
K1SPLIT = 79

_MESH = plsc.VectorSubcoreMesh(core_axis_name="c", subcore_axis_name="s")


def _zero_1d_slice(pay, dst_sh, off):
    for k in range(CPT // CH):
        pltpu.sync_copy(pay, dst_sh.at[pl.ds(off + k * CH, CH)])
    rem = CPT - (CPT // CH) * CH
    pltpu.sync_copy(pay.at[pl.ds(0, rem)],
                    dst_sh.at[pl.ds(off + (CPT // CH) * CH, rem)])


@functools.partial(
    pl.kernel,
    out_type=jax.ShapeDtypeStruct((NC, 2, NN), jnp.float32),
    mesh=_MESH,
    scratch_types=[
        pltpu.VMEM((NCHK, CH), jnp.int32),
        pltpu.VMEM((NCHK, CH), jnp.int32),
        pltpu.VMEM((CH,), jnp.float32),
        pltpu.SemaphoreType.DMA,
        pltpu.VMEM_SHARED((NN,), jnp.float32),
        pltpu.VMEM_SHARED((NN,), jnp.float32),
    ],
)
def _deg_kernel(src_hbm, dst_hbm, out_hbm, src_v, dst_v, pay, semS,
                hout_sh, hin_sh):
    c = lax.axis_index("c")
    s = lax.axis_index("s")
    for k in range(CH // L):
        pay[pl.ds(k * L, L)] = jnp.zeros((L,), jnp.float32)
    _zero_1d_slice(pay, hout_sh, s * CPT)
    _zero_1d_slice(pay, hin_sh, s * CPT)
    pltpu.sync_copy(src_hbm.at[s], src_v)
    pltpu.sync_copy(dst_hbm.at[s], dst_v)
    plsc.subcore_barrier()
    for k in range(CH // L):
        pay[pl.ds(k * L, L)] = jnp.ones((L,), jnp.float32)

    def body(i, carry):
        j = K1SPLIT * c + i

        @pl.when((i >= 1) & (i - 1 < K1SPLIT) & (j - 1 < NCHK))
        def _():
            pltpu.make_async_copy(pay, hout_sh.at[src_v.at[0]], semS).wait()
            pltpu.make_async_copy(pay, hin_sh.at[dst_v.at[0]], semS).wait()

        @pl.when((i < K1SPLIT) & (j < NCHK))
        def _():
            pltpu.async_copy(pay, hout_sh.at[src_v.at[j]], semS, add=True)
            pltpu.async_copy(pay, hin_sh.at[dst_v.at[j]], semS, add=True)

        return carry

    lax.fori_loop(0, K1SPLIT + 1, body, 0)
    plsc.subcore_barrier()

    @pl.when(s == 0)
    def _():
        pltpu.sync_copy(hout_sh, out_hbm.at[c, 0])
        pltpu.sync_copy(hin_sh, out_hbm.at[c, 1])


def _k2_body(x_ref, w1_ref, deg_ref, h_ref, dd_ref):
    dp = deg_ref[0]
    od = dp[:, 0:1] + dp[:, 2:3]
    idg = dp[:, 1:2] + dp[:, 3:4]
    so = lax.rsqrt(jnp.maximum(od, 1.0))
    si = lax.rsqrt(jnp.maximum(idg, 1.0))
    xw = jnp.dot(x_ref[...], w1_ref[...], preferred_element_type=jnp.float32)
    hw = xw * so
    h_ref[0] = hw[:, :FH]
    h_ref[1] = hw[:, FH:]
    dd_ref[0] = jnp.concatenate([so, si], axis=1)


def _run_k2(x_pad, W1, deg4):
    return pl.pallas_call(
        _k2_body,
        grid=(NBLK,),
        in_specs=[
            pl.BlockSpec((128, F), lambda b: (b, 0)),
            pl.BlockSpec((F, F), lambda b: (0, 0)),
            pl.BlockSpec((1, 128, 4), lambda b: (b, 0, 0)),
        ],
        out_specs=[
            pl.BlockSpec((NC, 128, FH), lambda b: (0, b, 0)),
            pl.BlockSpec((1, 128, 2), lambda b: (b, 0, 0)),
        ],
        out_shape=[
            jax.ShapeDtypeStruct((NC, NN, FH), jnp.float32),
            jax.ShapeDtypeStruct((NBLK, 128, 2), jnp.float32),
        ],
    )(x_pad, W1, deg4)


@functools.partial(
    pl.kernel,
    out_type=[
        jax.ShapeDtypeStruct((NC, NN, FH), jnp.float32),
        jax.ShapeDtypeStruct((NC, NN), jnp.float32),
    ],
    mesh=_MESH,
    scratch_types=[
        pltpu.VMEM((NCHK, CH), jnp.int32),
        pltpu.VMEM((NCHK, CH), jnp.int32),
        pltpu.VMEM((CH, FH), jnp.float32),
        pltpu.VMEM((CH, FH), jnp.float32),
        pltpu.VMEM((CH,), jnp.float32),
        pltpu.SemaphoreType.DMA,
        pltpu.SemaphoreType.DMA,
        pltpu.SemaphoreType.DMA,
        pltpu.SemaphoreType.DMA,
        pltpu.SemaphoreType.DMA,
        pltpu.VMEM_SHARED((NN, FH), jnp.float32),
        pltpu.VMEM_SHARED((NN,), jnp.float32),
    ],
    compiler_params=pltpu.CompilerParams(use_tc_tiling_on_sc=False),
)
def _mp_kernel(src_hbm, dst_hbm, din_hbm, h_hbm, agg_hbm, c_hbm,
               src_v, dst_v, rows0, rows1, cpay,
               semr0, semr1, semw0, semw1, semx, agg_sh, c_sh):
    c = lax.axis_index("c")
    s = lax.axis_index("s")

    for k in range(CH // L):
        cpay[pl.ds(k * L, L)] = jnp.zeros((L,), jnp.float32)
    _zero_1d_slice(cpay, c_sh, s * CPT)

    def zrow(i, carry):
        for k in range(FH // L):
            rows0[i, pl.ds(k * L, L)] = jnp.zeros((L,), jnp.float32)
        return carry

    lax.fori_loop(0, CH, zrow, 0)
    roff = s * CPT
    for k in range(CPT // CH):
        pltpu.sync_copy(rows0, agg_sh.at[pl.ds(roff + k * CH, CH)])
    rem = CPT - (CPT // CH) * CH
    pltpu.sync_copy(rows0.at[pl.ds(0, rem)],
                    agg_sh.at[pl.ds(roff + (CPT // CH) * CH, rem)])

    pltpu.sync_copy(src_hbm.at[s], src_v)
    pltpu.sync_copy(dst_hbm.at[s], dst_v)
    plsc.subcore_barrier()

    hsl = h_hbm.at[c]

    pltpu.async_copy(hsl.at[src_v.at[0]], rows0, semr0)
    pltpu.async_copy(hsl.at[src_v.at[1]], rows1, semr1)

    def body(i, carry):
        j0 = 2 * i
        j1 = j0 + 1
        jc = j0 + c
        pltpu.make_async_copy(hsl.at[src_v.at[j0]], rows0, semr0).wait()
        pltpu.async_copy(rows0, agg_sh.at[dst_v.at[j0]], semw0, add=True)

        @pl.when(i >= 1)
        def _():
            pltpu.make_async_copy(cpay, c_sh.at[src_v.at[0]], semx).wait()

        pltpu.sync_copy(din_hbm.at[dst_v.at[jc]], cpay)
        pltpu.async_copy(cpay, c_sh.at[src_v.at[jc]], semx, add=True)

        pltpu.make_async_copy(rows0, agg_sh.at[dst_v.at[j0]], semw0).wait()

        @pl.when(i < NCHK // 2 - 1)
        def _():
            pltpu.async_copy(hsl.at[src_v.at[j0 + 2]], rows0, semr0)

        @pl.when(i == NCHK // 2 - 1)
        def _():
            pltpu.async_copy(hsl.at[src_v.at[NCHK - 1]], rows0, semr0)

        pltpu.make_async_copy(hsl.at[src_v.at[j1]], rows1, semr1).wait()
        pltpu.async_copy(rows1, agg_sh.at[dst_v.at[j1]], semw1, add=True)
        pltpu.make_async_copy(rows1, agg_sh.at[dst_v.at[j1]], semw1).wait()

        @pl.when(i < NCHK // 2 - 1)
        def _():
            pltpu.async_copy(hsl.at[src_v.at[j1 + 2]], rows1, semr1)

        return carry

    lax.fori_loop(0, NCHK // 2, body, 0)
    jt = NCHK - 1
    pltpu.make_async_copy(hsl.at[src_v.at[jt]], rows0, semr0).wait()
    pltpu.sync_copy(rows0, agg_sh.at[dst_v.at[jt]], add=True)
    pltpu.make_async_copy(cpay, c_sh.at[src_v.at[0]], semx).wait()

    @pl.when(c == 0)
    def _():
        pltpu.sync_copy(din_hbm.at[dst_v.at[jt]], cpay)
        pltpu.sync_copy(cpay, c_sh.at[src_v.at[jt]], add=True)

    plsc.subcore_barrier()
    pltpu.sync_copy(agg_sh.at[pl.ds(roff, CPT)],
                    agg_hbm.at[c].at[pl.ds(roff, CPT)])

    @pl.when(s == 0)
    def _():
        pltpu.sync_copy(c_sh, c_hbm.at[c])


def _k4_body(agg_ref, cp_ref, dd_ref, b1_ref, w2_ref, b2_ref, out_ref, s_acc):
    b = pl.program_id(0)
    dp = dd_ref[0]
    so = dp[:, 0:1]
    si = dp[:, 1:2]
    a = jnp.concatenate([agg_ref[0], agg_ref[1]], axis=1)
    cp = cp_ref[0]
    cw = cp[:, 0:1] + cp[:, 1:2]
    m = jnp.maximum(a * si + b1_ref[...], 0.0)
    w = cw * so
    gid = lax.broadcasted_iota(jnp.int32, (128, 1), 0) + b * 128
    w = jnp.where(gid < N, w, 0.0)
    contrib = lax.dot_general(w, m, (((0,), (0,)), ((), ())),
                              preferred_element_type=jnp.float32)

    @pl.when(b == 0)
    def _():
        s_acc[...] = contrib

    @pl.when(b > 0)
    def _():
        s_acc[...] = s_acc[...] + contrib

    @pl.when(b == NBLK - 1)
    def _():
        out_ref[...] = (
            jnp.dot(s_acc[...] * (1.0 / N), w2_ref[...],
                    preferred_element_type=jnp.float32) + b2_ref[...]
        )


def _run_k4(agg_parts, c_parts3, dd, b1, W2, b2):
    return pl.pallas_call(
        _k4_body,
        grid=(NBLK,),
        in_specs=[
            pl.BlockSpec((NC, 128, FH), lambda b: (0, b, 0)),
            pl.BlockSpec((1, 128, NC), lambda b: (b, 0, 0)),
            pl.BlockSpec((1, 128, 2), lambda b: (b, 0, 0)),
            pl.BlockSpec((1, F), lambda b: (0, 0)),
            pl.BlockSpec((F, NCLS), lambda b: (0, 0)),
            pl.BlockSpec((1, NCLS), lambda b: (0, 0)),
        ],
        out_specs=pl.BlockSpec((1, NCLS), lambda b: (0, 0)),
        out_shape=jax.ShapeDtypeStruct((1, NCLS), jnp.float32),
        scratch_shapes=[pltpu.VMEM((1, F), jnp.float32)],
    )(agg_parts, c_parts3, dd, b1, W2, b2)


def kernel(in_feat, edge_index, W1, b1, W2, b2):
    ei = edge_index.astype(jnp.int32)
    sd = ei.reshape(2, NS, PT)
    pad = jnp.full((2, NS, PTP - PT), PAD_NODE, jnp.int32)
    sd = jnp.concatenate([sd, pad], axis=2).reshape(2, NS, NCHK, CH)
    src_slabs, dst_slabs = sd[0], sd[1]

    deg = _deg_kernel(src_slabs, dst_slabs)
    deg4 = deg.reshape(NC, 2, NBLK, 128).transpose(2, 3, 0, 1).reshape(
        NBLK, 128, NC * 2)

    x_pad = jnp.concatenate(
        [in_feat, jnp.zeros((NN - N, F), jnp.float32)], axis=0)
    h3, dd = _run_k2(x_pad, W1, deg4)

    din_flat = dd[:, :, 1].reshape(NN)
    agg_parts, c_parts = _mp_kernel(src_slabs, dst_slabs, din_flat, h3)

    c_parts3 = jnp.transpose(c_parts.reshape(NC, NBLK, 128), (1, 2, 0))
    return _run_k4(agg_parts, c_parts3, dd,
                   b1.reshape(1, F), W2, b2.reshape(1, NCLS))


# speedup vs baseline: 10.1622x; 1.0086x over previous
import functools

import jax
import jax.numpy as jnp
from jax import lax
from jax.experimental import pallas as pl
from jax.experimental.pallas import tpu as pltpu
from jax.experimental.pallas import tpu_sc as plsc

N = 10000
NN = 10112
E = 320000
F = 128
FH = F // 2
NCLS = 40
NC = 2
NS = 16
L = 16
CH = 128
PT = E // NS
NCHK = 157
PTP = NCHK * CH
NBLK = NN // 128
PAD_NODE = N
CPT = NN // NS
K1SPLIT = 79

_MESH = plsc.VectorSubcoreMesh(core_axis_name="c", subcore_axis_name="s")


def _zero_1d_slice(pay, dst_sh, off):
    for k in range(CPT // CH):
        pltpu.sync_copy(pay, dst_sh.at[pl.ds(off + k * CH, CH)])
    rem = CPT - (CPT // CH) * CH
    pltpu.sync_copy(pay.at[pl.ds(0, rem)],
                    dst_sh.at[pl.ds(off + (CPT // CH) * CH, rem)])


@functools.partial(
    pl.kernel,
    out_type=jax.ShapeDtypeStruct((NC, 2, NN), jnp.float32),
    mesh=_MESH,
    scratch_types=[
        pltpu.VMEM((NCHK, CH), jnp.int32),
        pltpu.VMEM((NCHK, CH), jnp.int32),
        pltpu.VMEM((CH,), jnp.float32),
        pltpu.SemaphoreType.DMA,
        pltpu.VMEM_SHARED((NN,), jnp.float32),
        pltpu.VMEM_SHARED((NN,), jnp.float32),
    ],
)
def _deg_kernel(src_hbm, dst_hbm, out_hbm, src_v, dst_v, pay, semS,
                hout_sh, hin_sh):
    c = lax.axis_index("c")
    s = lax.axis_index("s")
    for k in range(CH // L):
        pay[pl.ds(k * L, L)] = jnp.zeros((L,), jnp.float32)
    _zero_1d_slice(pay, hout_sh, s * CPT)
    _zero_1d_slice(pay, hin_sh, s * CPT)
    pltpu.sync_copy(src_hbm.at[s], src_v)
    pltpu.sync_copy(dst_hbm.at[s], dst_v)
    plsc.subcore_barrier()
    for k in range(CH // L):
        pay[pl.ds(k * L, L)] = jnp.ones((L,), jnp.float32)

    def body(i, carry):
        j = K1SPLIT * c + i

        @pl.when((i >= 1) & (i - 1 < K1SPLIT) & (j - 1 < NCHK))
        def _():
            pltpu.make_async_copy(pay, hout_sh.at[src_v.at[0]], semS).wait()
            pltpu.make_async_copy(pay, hin_sh.at[dst_v.at[0]], semS).wait()

        @pl.when((i < K1SPLIT) & (j < NCHK))
        def _():
            pltpu.async_copy(pay, hout_sh.at[src_v.at[j]], semS, add=True)
            pltpu.async_copy(pay, hin_sh.at[dst_v.at[j]], semS, add=True)

        return carry

    lax.fori_loop(0, K1SPLIT + 1, body, 0)
    plsc.subcore_barrier()

    @pl.when(s == 0)
    def _():
        pltpu.sync_copy(hout_sh, out_hbm.at[c, 0])
        pltpu.sync_copy(hin_sh, out_hbm.at[c, 1])


def _k2_body(x_ref, w1_ref, deg_ref, h_ref, dd_ref):
    dp = deg_ref[0]
    od = dp[:, 0:1] + dp[:, 2:3]
    idg = dp[:, 1:2] + dp[:, 3:4]
    so = lax.rsqrt(jnp.maximum(od, 1.0))
    si = lax.rsqrt(jnp.maximum(idg, 1.0))
    xw = jnp.dot(x_ref[...], w1_ref[...], preferred_element_type=jnp.float32)
    hw = (xw * so).astype(jnp.bfloat16)
    h_ref[0] = hw[:, :FH]
    h_ref[1] = hw[:, FH:]
    dd_ref[0] = jnp.concatenate([so, si], axis=1)


def _run_k2(x_pad, W1, deg4):
    return pl.pallas_call(
        _k2_body,
        grid=(NBLK,),
        in_specs=[
            pl.BlockSpec((128, F), lambda b: (b, 0)),
            pl.BlockSpec((F, F), lambda b: (0, 0)),
            pl.BlockSpec((1, 128, 4), lambda b: (b, 0, 0)),
        ],
        out_specs=[
            pl.BlockSpec((NC, 128, FH), lambda b: (0, b, 0)),
            pl.BlockSpec((1, 128, 2), lambda b: (b, 0, 0)),
        ],
        out_shape=[
            jax.ShapeDtypeStruct((NC, NN, FH), jnp.bfloat16),
            jax.ShapeDtypeStruct((NBLK, 128, 2), jnp.float32),
        ],
    )(x_pad, W1, deg4)


@functools.partial(
    pl.kernel,
    out_type=[
        jax.ShapeDtypeStruct((NC, 2, NN, FH), jnp.bfloat16),
        jax.ShapeDtypeStruct((NC, NN), jnp.float32),
    ],
    mesh=_MESH,
    scratch_types=[
        pltpu.VMEM((NCHK, CH), jnp.int32),
        pltpu.VMEM((NCHK, CH), jnp.int32),
        pltpu.VMEM((CH, FH), jnp.bfloat16),
        pltpu.VMEM((CH, FH), jnp.bfloat16),
        pltpu.VMEM((CH,), jnp.float32),
        pltpu.SemaphoreType.DMA,
        pltpu.SemaphoreType.DMA,
        pltpu.SemaphoreType.DMA,
        pltpu.SemaphoreType.DMA,
        pltpu.SemaphoreType.DMA,
        pltpu.VMEM_SHARED((NN, FH), jnp.bfloat16),
        pltpu.VMEM_SHARED((NN, FH), jnp.bfloat16),
        pltpu.VMEM_SHARED((NN,), jnp.float32),
    ],
    compiler_params=pltpu.CompilerParams(use_tc_tiling_on_sc=False),
)
def _mp_kernel(src_hbm, dst_hbm, din_hbm, h_hbm, agg_hbm, c_hbm,
               src_v, dst_v, rows0, rows1, cpay,
               semr0, semr1, semw0, semw1, semx, aggA_sh, aggB_sh, c_sh):
    c = lax.axis_index("c")
    s = lax.axis_index("s")

    for k in range(CH // L):
        cpay[pl.ds(k * L, L)] = jnp.zeros((L,), jnp.float32)
    _zero_1d_slice(cpay, c_sh, s * CPT)

    def zrow(i, carry):
        for k in range(FH // (2 * L)):
            rows0[i, pl.ds(k * 2 * L, 2 * L)] = jnp.zeros(
                (2 * L,), jnp.bfloat16)
        return carry

    lax.fori_loop(0, CH, zrow, 0)
    roff = s * CPT
    for k in range(CPT // CH):
        pltpu.sync_copy(rows0, aggA_sh.at[pl.ds(roff + k * CH, CH)])
        pltpu.sync_copy(rows0, aggB_sh.at[pl.ds(roff + k * CH, CH)])
    rem = CPT - (CPT // CH) * CH
    pltpu.sync_copy(rows0.at[pl.ds(0, rem)],
                    aggA_sh.at[pl.ds(roff + (CPT // CH) * CH, rem)])
    pltpu.sync_copy(rows0.at[pl.ds(0, rem)],
                    aggB_sh.at[pl.ds(roff + (CPT // CH) * CH, rem)])

    pltpu.sync_copy(src_hbm.at[s], src_v)
    pltpu.sync_copy(dst_hbm.at[s], dst_v)
    plsc.subcore_barrier()

    hsl = h_hbm.at[c]

    pltpu.async_copy(hsl.at[src_v.at[0]], rows0, semr0)
    pltpu.async_copy(hsl.at[src_v.at[1]], rows1, semr1)

    def body(i, carry):
        j0 = 2 * i
        j1 = j0 + 1
        jc = j0 + c
        pltpu.make_async_copy(hsl.at[src_v.at[j0]], rows0, semr0).wait()
        pltpu.async_copy(rows0, aggA_sh.at[dst_v.at[j0]], semw0, add=True)
        pltpu.make_async_copy(hsl.at[src_v.at[j1]], rows1, semr1).wait()
        pltpu.async_copy(rows1, aggB_sh.at[dst_v.at[j1]], semw1, add=True)

        @pl.when(i >= 1)
        def _():
            pltpu.make_async_copy(cpay, c_sh.at[src_v.at[0]], semx).wait()

        pltpu.sync_copy(din_hbm.at[dst_v.at[jc]], cpay)
        pltpu.async_copy(cpay, c_sh.at[src_v.at[jc]], semx, add=True)

        pltpu.make_async_copy(rows0, aggA_sh.at[dst_v.at[j0]], semw0).wait()

        @pl.when(i < NCHK // 2 - 1)
        def _():
            pltpu.async_copy(hsl.at[src_v.at[j0 + 2]], rows0, semr0)

        @pl.when(i == NCHK // 2 - 1)
        def _():
            pltpu.async_copy(hsl.at[src_v.at[NCHK - 1]], rows0, semr0)

        pltpu.make_async_copy(rows1, aggB_sh.at[dst_v.at[j1]], semw1).wait()

        @pl.when(i < NCHK // 2 - 1)
        def _():
            pltpu.async_copy(hsl.at[src_v.at[j1 + 2]], rows1, semr1)

        return carry

    lax.fori_loop(0, NCHK // 2, body, 0)
    jt = NCHK - 1
    pltpu.make_async_copy(hsl.at[src_v.at[jt]], rows0, semr0).wait()
    pltpu.sync_copy(rows0, aggA_sh.at[dst_v.at[jt]], add=True)
    pltpu.make_async_copy(cpay, c_sh.at[src_v.at[0]], semx).wait()

    @pl.when(c == 0)
    def _():
        pltpu.sync_copy(din_hbm.at[dst_v.at[jt]], cpay)
        pltpu.sync_copy(cpay, c_sh.at[src_v.at[jt]], add=True)

    plsc.subcore_barrier()
    pltpu.sync_copy(aggA_sh.at[pl.ds(roff, CPT)],
                    agg_hbm.at[c, 0].at[pl.ds(roff, CPT)])
    pltpu.sync_copy(aggB_sh.at[pl.ds(roff, CPT)],
                    agg_hbm.at[c, 1].at[pl.ds(roff, CPT)])

    @pl.when(s == 0)
    def _():
        pltpu.sync_copy(c_sh, c_hbm.at[c])


def _k4_body(agg_ref, cp_ref, dd_ref, b1_ref, w2_ref, b2_ref, out_ref, s_acc):
    b = pl.program_id(0)
    dp = dd_ref[0]
    so = dp[:, 0:1]
    si = dp[:, 1:2]
    a0 = (agg_ref[0, 0].astype(jnp.float32)
          + agg_ref[0, 1].astype(jnp.float32))
    a1 = (agg_ref[1, 0].astype(jnp.float32)
          + agg_ref[1, 1].astype(jnp.float32))
    a = jnp.concatenate([a0, a1], axis=1)
    cp = cp_ref[0]
    cw = cp[:, 0:1] + cp[:, 1:2]
    m = jnp.maximum(a * si + b1_ref[...], 0.0)
    w = cw * so
    gid = lax.broadcasted_iota(jnp.int32, (128, 1), 0) + b * 128
    w = jnp.where(gid < N, w, 0.0)
    m = jnp.where(gid < N, m, 0.0)
    contrib = lax.dot_general(w, m, (((0,), (0,)), ((), ())),
                              preferred_element_type=jnp.float32)

    @pl.when(b == 0)
    def _():
        s_acc[...] = contrib

    @pl.when(b > 0)
    def _():
        s_acc[...] = s_acc[...] + contrib

    @pl.when(b == NBLK - 1)
    def _():
        out_ref[...] = (
            jnp.dot(s_acc[...] * (1.0 / N), w2_ref[...],
                    preferred_element_type=jnp.float32) + b2_ref[...]
        )


def _run_k4(agg_parts, c_parts3, dd, b1, W2, b2):
    return pl.pallas_call(
        _k4_body,
        grid=(NBLK,),
        in_specs=[
            pl.BlockSpec((NC, 2, 128, FH), lambda b: (0, 0, b, 0)),
            pl.BlockSpec((1, 128, NC), lambda b: (b, 0, 0)),
            pl.BlockSpec((1, 128, 2), lambda b: (b, 0, 0)),
            pl.BlockSpec((1, F), lambda b: (0, 0)),
            pl.BlockSpec((F, NCLS), lambda b: (0, 0)),
            pl.BlockSpec((1, NCLS), lambda b: (0, 0)),
        ],
        out_specs=pl.BlockSpec((1, NCLS), lambda b: (0, 0)),
        out_shape=jax.ShapeDtypeStruct((1, NCLS), jnp.float32),
        scratch_shapes=[pltpu.VMEM((1, F), jnp.float32)],
    )(agg_parts, c_parts3, dd, b1, W2, b2)


def kernel(in_feat, edge_index, W1, b1, W2, b2):
    ei = edge_index.astype(jnp.int32)
    sd = ei.reshape(2, NS, PT)
    pad = jnp.full((2, NS, PTP - PT), PAD_NODE, jnp.int32)
    sd = jnp.concatenate([sd, pad], axis=2).reshape(2, NS, NCHK, CH)
    src_slabs, dst_slabs = sd[0], sd[1]

    deg = _deg_kernel(src_slabs, dst_slabs)
    deg4 = deg.reshape(NC, 2, NBLK, 128).transpose(2, 3, 0, 1).reshape(
        NBLK, 128, NC * 2)

    h3, dd = _run_k2(in_feat, W1, deg4)

    din_flat = dd[:, :, 1].reshape(NN)
    agg_parts, c_parts = _mp_kernel(src_slabs, dst_slabs, din_flat, h3)

    c_parts3 = jnp.transpose(c_parts.reshape(NC, NBLK, 128), (1, 2, 0))
    return _run_k4(agg_parts, c_parts3, dd,
                   b1.reshape(1, F), W2, b2.reshape(1, NCLS))


# speedup vs baseline: 12.2185x; 1.2023x over previous
import functools

import jax
import jax.numpy as jnp
from jax import lax
from jax.experimental import pallas as pl
from jax.experimental.pallas import tpu as pltpu
from jax.experimental.pallas import tpu_sc as plsc

N = 10000
NN = 10240
E = 320000
F = 128
FH = F // 2
NCLS = 40
NC = 2
NS = 16
L = 16
CH = 128
PT = E // NS
NCHK = 160
PTP = NCHK * CH
PAD_NODE = N
CPT = NN // NS
K1SPLIT = NCHK // NC
RB = NN // 8
G4 = NN // RB

_MESH = plsc.VectorSubcoreMesh(core_axis_name="c", subcore_axis_name="s")


def _zero_1d_slice(pay, dst_sh, off):
    for k in range(CPT // CH):
        pltpu.sync_copy(pay, dst_sh.at[pl.ds(off + k * CH, CH)])


@functools.partial(
    pl.kernel,
    out_type=jax.ShapeDtypeStruct((NC, 2, NN), jnp.float32),
    mesh=_MESH,
    scratch_types=[
        pltpu.VMEM((CH,), jnp.int32),
        pltpu.VMEM((CH,), jnp.int32),
        pltpu.VMEM((CH,), jnp.int32),
        pltpu.VMEM((CH,), jnp.int32),
        pltpu.VMEM((CH,), jnp.float32),
        pltpu.SemaphoreType.DMA,
        pltpu.SemaphoreType.DMA,
        pltpu.SemaphoreType.DMA,
        pltpu.VMEM_SHARED((NN,), jnp.float32),
        pltpu.VMEM_SHARED((NN,), jnp.float32),
    ],
)
def _deg_kernel(src_hbm, dst_hbm, out_hbm, sb0, sb1, db0, db1, pay,
                semA, semB, semS, hout_sh, hin_sh):
    c = lax.axis_index("c")
    s = lax.axis_index("s")
    for k in range(CH // L):
        pay[pl.ds(k * L, L)] = jnp.zeros((L,), jnp.float32)
    _zero_1d_slice(pay, hout_sh, s * CPT)
    _zero_1d_slice(pay, hin_sh, s * CPT)
    plsc.subcore_barrier()
    for k in range(CH // L):
        pay[pl.ds(k * L, L)] = jnp.ones((L,), jnp.float32)
    j00 = K1SPLIT * c
    pltpu.async_copy(src_hbm.at[s, j00], sb0, semA)
    pltpu.async_copy(dst_hbm.at[s, j00], db0, semA)

    def body(i, carry):
        j0 = j00 + 2 * i
        j1 = j0 + 1
        pltpu.make_async_copy(src_hbm.at[s, j0], sb0, semA).wait()
        pltpu.make_async_copy(dst_hbm.at[s, j0], db0, semA).wait()

        @pl.when(i >= 1)
        def _():
            pltpu.make_async_copy(pay, hout_sh.at[sb1], semS).wait()
            pltpu.make_async_copy(pay, hin_sh.at[db1], semS).wait()

        pltpu.async_copy(src_hbm.at[s, j1], sb1, semB)
        pltpu.async_copy(dst_hbm.at[s, j1], db1, semB)
        pltpu.async_copy(pay, hout_sh.at[sb0], semS, add=True)
        pltpu.async_copy(pay, hin_sh.at[db0], semS, add=True)
        pltpu.make_async_copy(src_hbm.at[s, j1], sb1, semB).wait()
        pltpu.make_async_copy(dst_hbm.at[s, j1], db1, semB).wait()
        pltpu.make_async_copy(pay, hout_sh.at[sb0], semS).wait()
        pltpu.make_async_copy(pay, hin_sh.at[db0], semS).wait()

        @pl.when(i < K1SPLIT // 2 - 1)
        def _():
            pltpu.async_copy(src_hbm.at[s, j0 + 2], sb0, semA)
            pltpu.async_copy(dst_hbm.at[s, j0 + 2], db0, semA)

        pltpu.async_copy(pay, hout_sh.at[sb1], semS, add=True)
        pltpu.async_copy(pay, hin_sh.at[db1], semS, add=True)
        return carry

    lax.fori_loop(0, K1SPLIT // 2, body, 0)
    pltpu.make_async_copy(pay, hout_sh.at[sb1], semS).wait()
    pltpu.make_async_copy(pay, hin_sh.at[db1], semS).wait()
    plsc.subcore_barrier()

    @pl.when(s == 0)
    def _():
        pltpu.sync_copy(hout_sh, out_hbm.at[c, 0])
        pltpu.sync_copy(hin_sh, out_hbm.at[c, 1])


def _k2_body(x_ref, w1_ref, deg_ref, h_ref, dd_ref):
    dp = deg_ref[...]
    od = dp[:, 0:1] + dp[:, 2:3]
    idg = dp[:, 1:2] + dp[:, 3:4]
    so = lax.rsqrt(jnp.maximum(od, 1.0))
    si = lax.rsqrt(jnp.maximum(idg, 1.0))
    xw = jnp.dot(x_ref[...], w1_ref[...], preferred_element_type=jnp.float32)
    hw = (xw * so).astype(jnp.bfloat16)
    h_ref[0] = hw[:, :FH]
    h_ref[1] = hw[:, FH:]
    dd_ref[...] = jnp.concatenate([so, si], axis=1)


def _run_k2(x, W1, deg_nm):
    return pl.pallas_call(
        _k2_body,
        grid=(G4,),
        in_specs=[
            pl.BlockSpec((RB, F), lambda b: (b, 0)),
            pl.BlockSpec((F, F), lambda b: (0, 0)),
            pl.BlockSpec((RB, 4), lambda b: (b, 0)),
        ],
        out_specs=[
            pl.BlockSpec((NC, RB, FH), lambda b: (0, b, 0)),
            pl.BlockSpec((RB, 2), lambda b: (b, 0)),
        ],
        out_shape=[
            jax.ShapeDtypeStruct((NC, NN, FH), jnp.bfloat16),
            jax.ShapeDtypeStruct((NN, 2), jnp.float32),
        ],
    )(x, W1, deg_nm)


@functools.partial(
    pl.kernel,
    out_type=[
        jax.ShapeDtypeStruct((NC, 2, NN, FH), jnp.bfloat16),
        jax.ShapeDtypeStruct((NC, NN), jnp.float32),
    ],
    mesh=_MESH,
    scratch_types=[
        pltpu.VMEM((NCHK, CH), jnp.int32),
        pltpu.VMEM((NCHK, CH), jnp.int32),
        pltpu.VMEM((CH, FH), jnp.bfloat16),
        pltpu.VMEM((CH, FH), jnp.bfloat16),
        pltpu.VMEM((CH, FH), jnp.bfloat16),
        pltpu.VMEM((CH, FH), jnp.bfloat16),
        pltpu.VMEM((CH,), jnp.float32),
        pltpu.VMEM((CH,), jnp.float32),
        pltpu.SemaphoreType.DMA,
        pltpu.SemaphoreType.DMA,
        pltpu.SemaphoreType.DMA,
        pltpu.SemaphoreType.DMA,
        pltpu.SemaphoreType.DMA,
        pltpu.SemaphoreType.DMA,
        pltpu.SemaphoreType.DMA,
        pltpu.SemaphoreType.DMA,
        pltpu.SemaphoreType.DMA,
        pltpu.VMEM_SHARED((NN, FH), jnp.bfloat16),
        pltpu.VMEM_SHARED((NN, FH), jnp.bfloat16),
        pltpu.VMEM_SHARED((NN,), jnp.float32),
    ],
    compiler_params=pltpu.CompilerParams(use_tc_tiling_on_sc=False),
)
def _mp_kernel(src_hbm, dst_hbm, din_hbm, h_hbm, agg_hbm, c_hbm,
               src_v, dst_v, r0, r1, r2, r3, cpayP, cpayQ,
               g0, g1, g2, g3, semA, semB, semcP, semcQ, semx,
               aggA_sh, aggB_sh, c_sh):
    c = lax.axis_index("c")
    s = lax.axis_index("s")

    for k in range(CH // L):
        cpayP[pl.ds(k * L, L)] = jnp.zeros((L,), jnp.float32)
    _zero_1d_slice(cpayP, c_sh, s * CPT)

    def zrow(i, carry):
        for k in range(FH // (2 * L)):
            r0[i, pl.ds(k * 2 * L, 2 * L)] = jnp.zeros((2 * L,), jnp.bfloat16)
        return carry

    lax.fori_loop(0, CH, zrow, 0)
    roff = s * CPT
    for k in range(CPT // CH):
        pltpu.sync_copy(r0, aggA_sh.at[pl.ds(roff + k * CH, CH)])
        pltpu.sync_copy(r0, aggB_sh.at[pl.ds(roff + k * CH, CH)])

    pltpu.sync_copy(src_hbm.at[s], src_v)
    pltpu.sync_copy(dst_hbm.at[s], dst_v)
    plsc.subcore_barrier()

    hsl = h_hbm.at[c]
    rbufs = (r0, r1, r2, r3)
    gsems = (g0, g1, g2, g3)

    for k in range(4):
        pltpu.async_copy(hsl.at[src_v.at[k]], rbufs[k], gsems[k])
    pltpu.async_copy(din_hbm.at[dst_v.at[c]], cpayP, semcP)
    pltpu.async_copy(din_hbm.at[dst_v.at[2 + c]], cpayQ, semcQ)

    NI = NCHK // 4

    def body(i, carry):
        j0 = 4 * i
        jc0 = j0 + c
        jc1 = j0 + 2 + c
        pltpu.make_async_copy(hsl.at[src_v.at[j0]], r0, g0).wait()
        pltpu.async_copy(r0, aggA_sh.at[dst_v.at[j0]], semA, add=True)
        pltpu.make_async_copy(hsl.at[src_v.at[j0 + 1]], r1, g1).wait()
        pltpu.async_copy(r1, aggB_sh.at[dst_v.at[j0 + 1]], semB, add=True)

        @pl.when(i >= 1)
        def _():
            pltpu.make_async_copy(cpayQ, c_sh.at[src_v.at[0]], semx).wait()
            pltpu.async_copy(din_hbm.at[dst_v.at[jc1]], cpayQ, semcQ)

        pltpu.make_async_copy(din_hbm.at[dst_v.at[jc0]], cpayP, semcP).wait()
        pltpu.async_copy(cpayP, c_sh.at[src_v.at[jc0]], semx, add=True)

        pltpu.make_async_copy(r0, aggA_sh.at[dst_v.at[j0]], semA).wait()

        @pl.when(i < NI - 1)
        def _():
            pltpu.async_copy(hsl.at[src_v.at[j0 + 4]], r0, g0)

        pltpu.make_async_copy(hsl.at[src_v.at[j0 + 2]], r2, g2).wait()
        pltpu.async_copy(r2, aggA_sh.at[dst_v.at[j0 + 2]], semA, add=True)
        pltpu.make_async_copy(r1, aggB_sh.at[dst_v.at[j0 + 1]], semB).wait()

        @pl.when(i < NI - 1)
        def _():
            pltpu.async_copy(hsl.at[src_v.at[j0 + 5]], r1, g1)

        pltpu.make_async_copy(hsl.at[src_v.at[j0 + 3]], r3, g3).wait()
        pltpu.async_copy(r3, aggB_sh.at[dst_v.at[j0 + 3]], semB, add=True)

        pltpu.make_async_copy(cpayP, c_sh.at[src_v.at[0]], semx).wait()

        @pl.when(i < NI - 1)
        def _():
            pltpu.async_copy(din_hbm.at[dst_v.at[jc0 + 4]], cpayP, semcP)

        pltpu.make_async_copy(din_hbm.at[dst_v.at[jc1]], cpayQ, semcQ).wait()
        pltpu.async_copy(cpayQ, c_sh.at[src_v.at[jc1]], semx, add=True)

        pltpu.make_async_copy(r2, aggA_sh.at[dst_v.at[j0 + 2]], semA).wait()

        @pl.when(i < NI - 1)
        def _():
            pltpu.async_copy(hsl.at[src_v.at[j0 + 6]], r2, g2)

        pltpu.make_async_copy(r3, aggB_sh.at[dst_v.at[j0 + 3]], semB).wait()

        @pl.when(i < NI - 1)
        def _():
            pltpu.async_copy(hsl.at[src_v.at[j0 + 7]], r3, g3)

        return carry

    lax.fori_loop(0, NI, body, 0)
    pltpu.make_async_copy(cpayQ, c_sh.at[src_v.at[0]], semx).wait()

    plsc.subcore_barrier()
    pltpu.sync_copy(aggA_sh.at[pl.ds(roff, CPT)],
                    agg_hbm.at[c, 0].at[pl.ds(roff, CPT)])
    pltpu.sync_copy(aggB_sh.at[pl.ds(roff, CPT)],
                    agg_hbm.at[c, 1].at[pl.ds(roff, CPT)])

    @pl.when(s == 0)
    def _():
        pltpu.sync_copy(c_sh, c_hbm.at[c])


def _k4_body(agg_ref, cp_ref, dd_ref, b1_ref, w2_ref, b2_ref, out_ref, s_acc):
    b = pl.program_id(0)
    dp = dd_ref[...]
    so = dp[:, 0:1]
    si = dp[:, 1:2]
    a0 = (agg_ref[0, 0].astype(jnp.float32)
          + agg_ref[0, 1].astype(jnp.float32))
    a1 = (agg_ref[1, 0].astype(jnp.float32)
          + agg_ref[1, 1].astype(jnp.float32))
    a = jnp.concatenate([a0, a1], axis=1)
    cp = cp_ref[...]
    cw = cp[:, 0:1] + cp[:, 1:2]
    m = jnp.maximum(a * si + b1_ref[...], 0.0)
    w = cw * so
    gid = lax.broadcasted_iota(jnp.int32, (RB, 1), 0) + b * RB
    w = jnp.where(gid < N, w, 0.0)
    m = jnp.where(gid < N, m, 0.0)
    contrib = lax.dot_general(w, m, (((0,), (0,)), ((), ())),
                              preferred_element_type=jnp.float32)

    @pl.when(b == 0)
    def _():
        s_acc[...] = contrib

    @pl.when(b > 0)
    def _():
        s_acc[...] = s_acc[...] + contrib

    @pl.when(b == G4 - 1)
    def _():
        out_ref[...] = (
            jnp.dot(s_acc[...] * (1.0 / N), w2_ref[...],
                    preferred_element_type=jnp.float32) + b2_ref[...]
        )


def _run_k4(agg_parts, cp_nm, dd, b1, W2, b2):
    return pl.pallas_call(
        _k4_body,
        grid=(G4,),
        in_specs=[
            pl.BlockSpec((NC, 2, RB, FH), lambda b: (0, 0, b, 0)),
            pl.BlockSpec((RB, NC), lambda b: (b, 0)),
            pl.BlockSpec((RB, 2), lambda b: (b, 0)),
            pl.BlockSpec((1, F), lambda b: (0, 0)),
            pl.BlockSpec((F, NCLS), lambda b: (0, 0)),
            pl.BlockSpec((1, NCLS), lambda b: (0, 0)),
        ],
        out_specs=pl.BlockSpec((1, NCLS), lambda b: (0, 0)),
        out_shape=jax.ShapeDtypeStruct((1, NCLS), jnp.float32),
        scratch_shapes=[pltpu.VMEM((1, F), jnp.float32)],
    )(agg_parts, cp_nm, dd, b1, W2, b2)


def kernel(in_feat, edge_index, W1, b1, W2, b2):
    ei = edge_index.astype(jnp.int32)
    sd = ei.reshape(2, NS, PT)
    pad = jnp.full((2, NS, PTP - PT), PAD_NODE, jnp.int32)
    sd = jnp.concatenate([sd, pad], axis=2).reshape(2, NS, NCHK, CH)
    src_slabs, dst_slabs = sd[0], sd[1]

    deg = _deg_kernel(src_slabs, dst_slabs)
    deg_nm = jnp.transpose(deg.reshape(NC * 2, NN), (1, 0))

    h3, dd = _run_k2(in_feat, W1, deg_nm)

    din_flat = dd[:, 1]
    agg_parts, c_parts = _mp_kernel(src_slabs, dst_slabs, din_flat, h3)

    cp_nm = jnp.transpose(c_parts, (1, 0))
    return _run_k4(agg_parts, cp_nm, dd,
                   b1.reshape(1, F), W2, b2.reshape(1, NCLS))


# speedup vs baseline: 15.1066x; 1.2364x over previous
import functools

import jax
import jax.numpy as jnp
from jax import lax
from jax.experimental import pallas as pl
from jax.experimental.pallas import tpu as pltpu
from jax.experimental.pallas import tpu_sc as plsc

N = 10000
NN = 10240
E = 320000
F = 128
FH = F // 2
NCLS = 40
NC = 2
NS = 16
L = 16
CH = 128
PT = E // NS
NCHK = 160
PTP = NCHK * CH
PAD_NODE = N
CPT = NN // NS
K1SPLIT = NCHK // NC
K1BLK = 16
RB = NN // 8
G4 = NN // RB

_MESH = plsc.VectorSubcoreMesh(core_axis_name="c", subcore_axis_name="s")


def _zero_1d_slice(pay, dst_sh, off):
    for k in range(CPT // CH):
        pltpu.sync_copy(pay, dst_sh.at[pl.ds(off + k * CH, CH)])


@functools.partial(
    pl.kernel,
    out_type=jax.ShapeDtypeStruct((NC, 2, 2, NN), jnp.float32),
    mesh=_MESH,
    scratch_types=[
        pltpu.VMEM((K1BLK, CH), jnp.int32),
        pltpu.VMEM((K1BLK, CH), jnp.int32),
        pltpu.VMEM((CH,), jnp.float32),
        pltpu.SemaphoreType.DMA,
        pltpu.SemaphoreType.DMA,
        pltpu.VMEM_SHARED((NN,), jnp.float32),
        pltpu.VMEM_SHARED((NN,), jnp.float32),
        pltpu.VMEM_SHARED((NN,), jnp.float32),
        pltpu.VMEM_SHARED((NN,), jnp.float32),
    ],
)
def _deg_kernel(src_hbm, dst_hbm, out_hbm, sslab, dslab, pay,
                semSA, semSB, houtA, hinA, houtB, hinB):
    c = lax.axis_index("c")
    s = lax.axis_index("s")
    for k in range(CH // L):
        pay[pl.ds(k * L, L)] = jnp.zeros((L,), jnp.float32)
    _zero_1d_slice(pay, houtA, s * CPT)
    _zero_1d_slice(pay, hinA, s * CPT)
    _zero_1d_slice(pay, houtB, s * CPT)
    _zero_1d_slice(pay, hinB, s * CPT)
    plsc.subcore_barrier()
    for k in range(CH // L):
        pay[pl.ds(k * L, L)] = jnp.ones((L,), jnp.float32)
    j00 = K1SPLIT * c

    for st in range(K1SPLIT // K1BLK):
        if st > 0:
            pltpu.make_async_copy(pay, houtA.at[sslab.at[0]], semSA).wait()
            pltpu.make_async_copy(pay, hinA.at[dslab.at[0]], semSA).wait()
            pltpu.make_async_copy(pay, houtB.at[sslab.at[0]], semSB).wait()
            pltpu.make_async_copy(pay, hinB.at[dslab.at[0]], semSB).wait()
        pltpu.sync_copy(src_hbm.at[s].at[pl.ds(j00 + st * K1BLK, K1BLK)],
                        sslab)
        pltpu.sync_copy(dst_hbm.at[s].at[pl.ds(j00 + st * K1BLK, K1BLK)],
                        dslab)

        def inner(m, carry):
            jA = 2 * m
            jB = jA + 1

            @pl.when(m > 0)
            def _():
                pltpu.make_async_copy(pay, houtA.at[sslab.at[0]],
                                      semSA).wait()
                pltpu.make_async_copy(pay, hinA.at[dslab.at[0]],
                                      semSA).wait()

            pltpu.async_copy(pay, houtA.at[sslab.at[jA]], semSA, add=True)
            pltpu.async_copy(pay, hinA.at[dslab.at[jA]], semSA, add=True)

            @pl.when(m > 0)
            def _():
                pltpu.make_async_copy(pay, houtB.at[sslab.at[0]],
                                      semSB).wait()
                pltpu.make_async_copy(pay, hinB.at[dslab.at[0]],
                                      semSB).wait()

            pltpu.async_copy(pay, houtB.at[sslab.at[jB]], semSB, add=True)
            pltpu.async_copy(pay, hinB.at[dslab.at[jB]], semSB, add=True)
            return carry

        lax.fori_loop(0, K1BLK // 2, inner, 0)

    pltpu.make_async_copy(pay, houtA.at[sslab.at[0]], semSA).wait()
    pltpu.make_async_copy(pay, hinA.at[dslab.at[0]], semSA).wait()
    pltpu.make_async_copy(pay, houtB.at[sslab.at[0]], semSB).wait()
    pltpu.make_async_copy(pay, hinB.at[dslab.at[0]], semSB).wait()
    plsc.subcore_barrier()

    @pl.when(s == 0)
    def _():
        pltpu.sync_copy(houtA, out_hbm.at[c, 0, 0])
        pltpu.sync_copy(hinA, out_hbm.at[c, 1, 0])
        pltpu.sync_copy(houtB, out_hbm.at[c, 0, 1])
        pltpu.sync_copy(hinB, out_hbm.at[c, 1, 1])


def _k2_body(x_ref, w1_ref, deg_ref, h_ref, dd_ref):
    dp = deg_ref[...]
    od = dp[:, 0:1] + dp[:, 1:2] + dp[:, 4:5] + dp[:, 5:6]
    idg = dp[:, 2:3] + dp[:, 3:4] + dp[:, 6:7] + dp[:, 7:8]
    so = lax.rsqrt(jnp.maximum(od, 1.0))
    si = lax.rsqrt(jnp.maximum(idg, 1.0))
    xw = jnp.dot(x_ref[...], w1_ref[...], preferred_element_type=jnp.float32)
    hw = (xw * so).astype(jnp.bfloat16)
    h_ref[0] = hw[:, :FH]
    h_ref[1] = hw[:, FH:]
    dd_ref[...] = jnp.concatenate([so, si], axis=1)


def _run_k2(x, W1, deg_nm):
    return pl.pallas_call(
        _k2_body,
        grid=(G4,),
        in_specs=[
            pl.BlockSpec((RB, F), lambda b: (b, 0)),
            pl.BlockSpec((F, F), lambda b: (0, 0)),
            pl.BlockSpec((RB, 8), lambda b: (b, 0)),
        ],
        out_specs=[
            pl.BlockSpec((NC, RB, FH), lambda b: (0, b, 0)),
            pl.BlockSpec((RB, 2), lambda b: (b, 0)),
        ],
        out_shape=[
            jax.ShapeDtypeStruct((NC, NN, FH), jnp.bfloat16),
            jax.ShapeDtypeStruct((NN, 2), jnp.float32),
        ],
    )(x, W1, deg_nm)


@functools.partial(
    pl.kernel,
    out_type=[
        jax.ShapeDtypeStruct((NC, 2, NN, FH), jnp.bfloat16),
        jax.ShapeDtypeStruct((NC, NN), jnp.float32),
    ],
    mesh=_MESH,
    scratch_types=[
        pltpu.VMEM((NCHK, CH), jnp.int32),
        pltpu.VMEM((NCHK, CH), jnp.int32),
        pltpu.VMEM((CH, FH), jnp.bfloat16),
        pltpu.VMEM((CH, FH), jnp.bfloat16),
        pltpu.VMEM((CH, FH), jnp.bfloat16),
        pltpu.VMEM((CH, FH), jnp.bfloat16),
        pltpu.VMEM((CH,), jnp.float32),
        pltpu.VMEM((CH,), jnp.float32),
        pltpu.SemaphoreType.DMA,
        pltpu.SemaphoreType.DMA,
        pltpu.SemaphoreType.DMA,
        pltpu.SemaphoreType.DMA,
        pltpu.SemaphoreType.DMA,
        pltpu.SemaphoreType.DMA,
        pltpu.SemaphoreType.DMA,
        pltpu.SemaphoreType.DMA,
        pltpu.SemaphoreType.DMA,
        pltpu.VMEM_SHARED((NN, FH), jnp.bfloat16),
        pltpu.VMEM_SHARED((NN, FH), jnp.bfloat16),
        pltpu.VMEM_SHARED((NN, FH), jnp.bfloat16),
        pltpu.VMEM_SHARED((NN,), jnp.float32),
    ],
    compiler_params=pltpu.CompilerParams(use_tc_tiling_on_sc=False),
)
def _mp_kernel(src_hbm, dst_hbm, din_hbm, h_hbm, agg_hbm, c_hbm,
               src_v, dst_v, r0, r1, r2, r3, cpayP, cpayQ,
               g0, g1, g2, g3, semA, semB, semcP, semcQ, semx,
               aggA_sh, aggB_sh, h_sh, c_sh):
    c = lax.axis_index("c")
    s = lax.axis_index("s")

    for k in range(CH // L):
        cpayP[pl.ds(k * L, L)] = jnp.zeros((L,), jnp.float32)
    _zero_1d_slice(cpayP, c_sh, s * CPT)

    def zrow(i, carry):
        for k in range(FH // (2 * L)):
            r0[i, pl.ds(k * 2 * L, 2 * L)] = jnp.zeros((2 * L,), jnp.bfloat16)
        return carry

    lax.fori_loop(0, CH, zrow, 0)
    roff = s * CPT
    for k in range(CPT // CH):
        pltpu.sync_copy(r0, aggA_sh.at[pl.ds(roff + k * CH, CH)])
        pltpu.sync_copy(r0, aggB_sh.at[pl.ds(roff + k * CH, CH)])

    pltpu.sync_copy(src_hbm.at[s], src_v)
    pltpu.sync_copy(dst_hbm.at[s], dst_v)
    pltpu.sync_copy(h_hbm.at[c].at[pl.ds(roff, CPT)],
                    h_sh.at[pl.ds(roff, CPT)])
    plsc.subcore_barrier()

    hsl = h_sh
    rbufs = (r0, r1, r2, r3)
    gsems = (g0, g1, g2, g3)

    for k in range(4):
        pltpu.async_copy(hsl.at[src_v.at[k]], rbufs[k], gsems[k])
    pltpu.async_copy(din_hbm.at[dst_v.at[c]], cpayP, semcP)
    pltpu.async_copy(din_hbm.at[dst_v.at[2 + c]], cpayQ, semcQ)

    NI = NCHK // 4

    def body(i, carry):
        j0 = 4 * i
        jc0 = j0 + c
        jc1 = j0 + 2 + c
        pltpu.make_async_copy(hsl.at[src_v.at[j0]], r0, g0).wait()
        pltpu.async_copy(r0, aggA_sh.at[dst_v.at[j0]], semA, add=True)
        pltpu.make_async_copy(hsl.at[src_v.at[j0 + 1]], r1, g1).wait()
        pltpu.async_copy(r1, aggB_sh.at[dst_v.at[j0 + 1]], semB, add=True)

        @pl.when(i >= 1)
        def _():
            pltpu.make_async_copy(cpayQ, c_sh.at[src_v.at[0]], semx).wait()
            pltpu.async_copy(din_hbm.at[dst_v.at[jc1]], cpayQ, semcQ)

        pltpu.make_async_copy(din_hbm.at[dst_v.at[jc0]], cpayP, semcP).wait()
        pltpu.async_copy(cpayP, c_sh.at[src_v.at[jc0]], semx, add=True)

        pltpu.make_async_copy(r0, aggA_sh.at[dst_v.at[j0]], semA).wait()

        @pl.when(i < NI - 1)
        def _():
            pltpu.async_copy(hsl.at[src_v.at[j0 + 4]], r0, g0)

        pltpu.make_async_copy(hsl.at[src_v.at[j0 + 2]], r2, g2).wait()
        pltpu.async_copy(r2, aggA_sh.at[dst_v.at[j0 + 2]], semA, add=True)
        pltpu.make_async_copy(r1, aggB_sh.at[dst_v.at[j0 + 1]], semB).wait()

        @pl.when(i < NI - 1)
        def _():
            pltpu.async_copy(hsl.at[src_v.at[j0 + 5]], r1, g1)

        pltpu.make_async_copy(hsl.at[src_v.at[j0 + 3]], r3, g3).wait()
        pltpu.async_copy(r3, aggB_sh.at[dst_v.at[j0 + 3]], semB, add=True)

        pltpu.make_async_copy(cpayP, c_sh.at[src_v.at[0]], semx).wait()

        @pl.when(i < NI - 1)
        def _():
            pltpu.async_copy(din_hbm.at[dst_v.at[jc0 + 4]], cpayP, semcP)

        pltpu.make_async_copy(din_hbm.at[dst_v.at[jc1]], cpayQ, semcQ).wait()
        pltpu.async_copy(cpayQ, c_sh.at[src_v.at[jc1]], semx, add=True)

        pltpu.make_async_copy(r2, aggA_sh.at[dst_v.at[j0 + 2]], semA).wait()

        @pl.when(i < NI - 1)
        def _():
            pltpu.async_copy(hsl.at[src_v.at[j0 + 6]], r2, g2)

        pltpu.make_async_copy(r3, aggB_sh.at[dst_v.at[j0 + 3]], semB).wait()

        @pl.when(i < NI - 1)
        def _():
            pltpu.async_copy(hsl.at[src_v.at[j0 + 7]], r3, g3)

        return carry

    lax.fori_loop(0, NI, body, 0)
    pltpu.make_async_copy(cpayQ, c_sh.at[src_v.at[0]], semx).wait()

    plsc.subcore_barrier()
    pltpu.sync_copy(aggA_sh.at[pl.ds(roff, CPT)],
                    agg_hbm.at[c, 0].at[pl.ds(roff, CPT)])
    pltpu.sync_copy(aggB_sh.at[pl.ds(roff, CPT)],
                    agg_hbm.at[c, 1].at[pl.ds(roff, CPT)])

    @pl.when(s == 0)
    def _():
        pltpu.sync_copy(c_sh, c_hbm.at[c])


def _k4_body(agg_ref, cp_ref, dd_ref, b1_ref, w2_ref, b2_ref, out_ref, s_acc):
    b = pl.program_id(0)
    dp = dd_ref[...]
    so = dp[:, 0:1]
    si = dp[:, 1:2]
    a0 = (agg_ref[0, 0].astype(jnp.float32)
          + agg_ref[0, 1].astype(jnp.float32))
    a1 = (agg_ref[1, 0].astype(jnp.float32)
          + agg_ref[1, 1].astype(jnp.float32))
    a = jnp.concatenate([a0, a1], axis=1)
    cp = cp_ref[...]
    cw = cp[:, 0:1] + cp[:, 1:2]
    m = jnp.maximum(a * si + b1_ref[...], 0.0)
    w = cw * so
    gid = lax.broadcasted_iota(jnp.int32, (RB, 1), 0) + b * RB
    w = jnp.where(gid < N, w, 0.0)
    m = jnp.where(gid < N, m, 0.0)
    contrib = lax.dot_general(w, m, (((0,), (0,)), ((), ())),
                              preferred_element_type=jnp.float32)

    @pl.when(b == 0)
    def _():
        s_acc[...] = contrib

    @pl.when(b > 0)
    def _():
        s_acc[...] = s_acc[...] + contrib

    @pl.when(b == G4 - 1)
    def _():
        out_ref[...] = (
            jnp.dot(s_acc[...] * (1.0 / N), w2_ref[...],
                    preferred_element_type=jnp.float32) + b2_ref[...]
        )


def _run_k4(agg_parts, cp_nm, dd, b1, W2, b2):
    return pl.pallas_call(
        _k4_body,
        grid=(G4,),
        in_specs=[
            pl.BlockSpec((NC, 2, RB, FH), lambda b: (0, 0, b, 0)),
            pl.BlockSpec((RB, NC), lambda b: (b, 0)),
            pl.BlockSpec((RB, 2), lambda b: (b, 0)),
            pl.BlockSpec((1, F), lambda b: (0, 0)),
            pl.BlockSpec((F, NCLS), lambda b: (0, 0)),
            pl.BlockSpec((1, NCLS), lambda b: (0, 0)),
        ],
        out_specs=pl.BlockSpec((1, NCLS), lambda b: (0, 0)),
        out_shape=jax.ShapeDtypeStruct((1, NCLS), jnp.float32),
        scratch_shapes=[pltpu.VMEM((1, F), jnp.float32)],
    )(agg_parts, cp_nm, dd, b1, W2, b2)


def kernel(in_feat, edge_index, W1, b1, W2, b2):
    ei = edge_index.astype(jnp.int32)
    sd = ei.reshape(2, NS, PT)
    pad = jnp.full((2, NS, PTP - PT), PAD_NODE, jnp.int32)
    sd = jnp.concatenate([sd, pad], axis=2).reshape(2, NS, NCHK, CH)
    src_slabs, dst_slabs = sd[0], sd[1]

    deg = _deg_kernel(src_slabs, dst_slabs)
    deg_nm = jnp.transpose(deg.reshape(NC * 4, NN), (1, 0))

    h3, dd = _run_k2(in_feat, W1, deg_nm)

    din_flat = dd[:, 1]
    agg_parts, c_parts = _mp_kernel(src_slabs, dst_slabs, din_flat, h3)

    cp_nm = jnp.transpose(c_parts, (1, 0))
    return _run_k4(agg_parts, cp_nm, dd,
                   b1.reshape(1, F), W2, b2.reshape(1, NCLS))
